# Initial kernel scaffold; baseline (speedup 1.0000x reference)
import jax
import jax.numpy as jnp
from jax.experimental import pallas as pl


def kernel(x, Wg1, bg1, Wg2, bg2, We, be):
    raise NotImplementedError("write your pallas kernel here")


# speedup vs baseline: 1.6853x; 1.6853x over previous
import jax
import jax.numpy as jnp
from jax import lax
from jax.experimental import pallas as pl

_NEG = -1e30


def _moe_body(x_ref, Wg1_ref, bg1_ref, Wg2_ref, bg2_ref, We_ref, beT_ref,
              out_ref):
    xb = x_ref[0]
    C = xb.shape[0]
    E = We_ref.shape[0]
    pooled = jnp.mean(xb, axis=1, keepdims=True)
    h = lax.dot_general(Wg1_ref[...], pooled, (((1,), (0,)), ((), ())),
                        preferred_element_type=jnp.float32)
    h = jnp.maximum(h + bg1_ref[...], 0.0)
    logits = lax.dot_general(Wg2_ref[...], h, (((1,), (0,)), ((), ())),
                             preferred_element_type=jnp.float32)
    logits = logits + bg2_ref[...]
    lmax = jnp.max(logits)
    ex = jnp.exp(logits - lmax)
    probs = ex / jnp.sum(ex)
    eidx = lax.broadcasted_iota(jnp.int32, probs.shape, 0)
    m1 = jnp.max(probs)
    i1 = jnp.min(jnp.where(probs == m1, eidx, E))
    probs2 = jnp.where(eidx == i1, _NEG, probs)
    m2 = jnp.max(probs2)
    i2 = jnp.min(jnp.where(probs2 == m2, eidx, E))
    inv = 1.0 / (m1 + m2 + 1e-8)
    Wc = jnp.zeros(We_ref.shape[1:], dtype=jnp.float32)
    bc = jnp.zeros((C, 1), dtype=jnp.float32)
    for e in range(E):
        ge = ((i1 == e).astype(jnp.float32) * m1 +
              (i2 == e).astype(jnp.float32) * m2) * inv
        Wc = Wc + ge * We_ref[e]
        bc = bc + ge * beT_ref[:, e:e + 1]
    y = lax.dot_general(Wc, xb, (((1,), (0,)), ((), ())),
                        preferred_element_type=jnp.float32)
    out_ref[0] = y + bc + xb


def kernel(x, Wg1, bg1, Wg2, bg2, We, be):
    B, C, H, W = x.shape
    E, GH = Wg2.shape
    HW = H * W
    x3 = x.reshape(B, C, HW)
    out = pl.pallas_call(
        _moe_body,
        grid=(B,),
        in_specs=[
            pl.BlockSpec((1, C, HW), lambda b: (b, 0, 0)),
            pl.BlockSpec((GH, C), lambda b: (0, 0)),
            pl.BlockSpec((GH, 1), lambda b: (0, 0)),
            pl.BlockSpec((E, GH), lambda b: (0, 0)),
            pl.BlockSpec((E, 1), lambda b: (0, 0)),
            pl.BlockSpec((E, C, C), lambda b: (0, 0, 0)),
            pl.BlockSpec((C, E), lambda b: (0, 0)),
        ],
        out_specs=pl.BlockSpec((1, C, HW), lambda b: (b, 0, 0)),
        out_shape=jax.ShapeDtypeStruct((B, C, HW), jnp.float32),
    )(x3, Wg1, bg1.reshape(GH, 1), Wg2, bg2.reshape(E, 1), We, be.T)
    return out.reshape(B, C, H, W)


# speedup vs baseline: 1.9428x; 1.1528x over previous
import jax
import jax.numpy as jnp
from jax import lax
from jax.experimental import pallas as pl
from jax.experimental.pallas import tpu as pltpu


def _gate_body(x_ref, Wg1_ref, bg1_ref, Wg2_ref, bg2_ref, idx_ref, val_ref):
    E = bg2_ref.shape[1]
    pooled = jnp.mean(x_ref[...], axis=2)
    h = lax.dot_general(pooled, Wg1_ref[...], (((1,), (1,)), ((), ())),
                        preferred_element_type=jnp.float32)
    h = jnp.maximum(h + bg1_ref[...], 0.0)
    logits = lax.dot_general(h, Wg2_ref[...], (((1,), (1,)), ((), ())),
                             preferred_element_type=jnp.float32)
    logits = logits + bg2_ref[...]
    rmax = jnp.max(logits, axis=1, keepdims=True)
    ex = jnp.exp(logits - rmax)
    probs = ex / jnp.sum(ex, axis=1, keepdims=True)
    eidx = lax.broadcasted_iota(jnp.int32, probs.shape, 1)
    m1 = jnp.max(probs, axis=1, keepdims=True)
    i1 = jnp.min(jnp.where(probs == m1, eidx, E), axis=1, keepdims=True)
    probs2 = jnp.where(eidx == i1, -1.0, probs)
    m2 = jnp.max(probs2, axis=1, keepdims=True)
    i2 = jnp.min(jnp.where(probs2 == m2, eidx, E), axis=1, keepdims=True)
    inv = 1.0 / (m1 + m2 + 1e-8)
    idx_ref[...] = jnp.concatenate([i1, i2], axis=1)
    val_ref[...] = jnp.concatenate([m1 * inv, m2 * inv], axis=1)


def _combine_body(idx_ref, val_ref, x_ref, We_ref, be3_ref, out_ref):
    i1 = idx_ref[0, 0, 0]
    i2 = idx_ref[0, 0, 1]
    v1 = val_ref[0, 0, 0]
    v2 = val_ref[0, 0, 1]
    xb = x_ref[0]
    S1 = We_ref[pl.ds(i1, 1), :, :][0]
    S2 = We_ref[pl.ds(i2, 1), :, :][0]
    Wc = v1 * S1 + v2 * S2
    bc = v1 * be3_ref[pl.ds(i1, 1), :, :][0] + v2 * be3_ref[pl.ds(i2, 1), :, :][0]
    y = lax.dot_general(Wc, xb, (((1,), (0,)), ((), ())),
                        preferred_element_type=jnp.float32)
    out_ref[0] = y + bc + xb


def kernel(x, Wg1, bg1, Wg2, bg2, We, be):
    B, C, H, W = x.shape
    E, GH = Wg2.shape
    HW = H * W
    x3 = x.reshape(B, C, HW)
    idx, val = pl.pallas_call(
        _gate_body,
        in_specs=[
            pl.BlockSpec((B, C, HW), lambda: (0, 0, 0)),
            pl.BlockSpec((GH, C), lambda: (0, 0)),
            pl.BlockSpec((1, GH), lambda: (0, 0)),
            pl.BlockSpec((E, GH), lambda: (0, 0)),
            pl.BlockSpec((1, E), lambda: (0, 0)),
        ],
        out_specs=[
            pl.BlockSpec((B, 2), lambda: (0, 0)),
            pl.BlockSpec((B, 2), lambda: (0, 0)),
        ],
        out_shape=[
            jax.ShapeDtypeStruct((B, 2), jnp.int32),
            jax.ShapeDtypeStruct((B, 2), jnp.float32),
        ],
    )(x3, Wg1, bg1.reshape(1, GH), Wg2, bg2.reshape(1, E))
    out = pl.pallas_call(
        _combine_body,
        grid=(B,),
        in_specs=[
            pl.BlockSpec((1, 1, 2), lambda b: (b, 0, 0), memory_space=pltpu.SMEM),
            pl.BlockSpec((1, 1, 2), lambda b: (b, 0, 0), memory_space=pltpu.SMEM),
            pl.BlockSpec((1, C, HW), lambda b: (b, 0, 0)),
            pl.BlockSpec((E, C, C), lambda b: (0, 0, 0)),
            pl.BlockSpec((E, C, 1), lambda b: (0, 0, 0)),
        ],
        out_specs=pl.BlockSpec((1, C, HW), lambda b: (b, 0, 0)),
        out_shape=jax.ShapeDtypeStruct((B, C, HW), jnp.float32),
    )(idx.reshape(B, 1, 2), val.reshape(B, 1, 2), x3, We, be[:, :, None])
    return out.reshape(B, C, H, W)
